# baseline (speedup 1.0000x reference)
import functools

import jax
import jax.numpy as jnp
from jax import lax
from jax.experimental import pallas as pl
from jax.experimental.pallas import tpu as pltpu
from jax.experimental.pallas import tpu_sc as plsc

_LANES = 16


@functools.cache
def _build_sc_kernel(n: int, num_types: int):
    info = plsc.get_sparse_core_info()
    nc, ns = info.num_cores, info.num_subcores
    nw = nc * ns
    chunk = ((n + nw - 1) // nw + _LANES - 1) // _LANES * _LANES
    assert n % 8 == 0 and chunk % 8 == 0 and n >= chunk

    mesh = plsc.VectorSubcoreMesh(core_axis_name="c", subcore_axis_name="s")

    @functools.partial(
        pl.kernel,
        mesh=mesh,
        out_type=jax.ShapeDtypeStruct((n, 1), jnp.float32),
        compiler_params=pltpu.CompilerParams(
            needs_layout_passes=False,
            disable_bounds_checks=True,
            disable_semaphore_checks=True,
            use_tc_tiling_on_sc=False,
        ),
        scratch_types=[
            pltpu.VMEM((chunk,), jnp.int32),
            pltpu.VMEM((chunk, 1), jnp.float32),
            pltpu.VMEM((num_types, 1), jnp.float32),
            pltpu.SemaphoreType.DMA,
            pltpu.SemaphoreType.DMA,
            pltpu.SemaphoreType.DMA,
        ],
    )
    def sc_kernel(e_hbm, atoms_hbm, refs_hbm, out_hbm, idx_v, e_v, tbl_v,
                  sem_i, sem_e, sem_t):
        wid = lax.axis_index("s") * nc + lax.axis_index("c")
        base = jnp.minimum(wid * chunk, n - chunk)
        base = pl.multiple_of(base, 8)
        cp_t = pltpu.async_copy(refs_hbm, tbl_v, sem_t)
        cp_i = pltpu.async_copy(atoms_hbm.at[pl.ds(base, chunk)], idx_v, sem_i)
        cp_e = pltpu.async_copy(e_hbm.at[pl.ds(base, chunk), :], e_v, sem_e)
        cp_t.wait()
        cp_i.wait()
        cp_e.wait()

        zero = jnp.zeros((_LANES,), jnp.int32)
        lane = lax.iota(jnp.int32, _LANES)

        @plsc.parallel_loop(0, chunk, _LANES, unroll=8)
        def _(i):
            ii = lane + i
            r16 = plsc.load_gather(tbl_v, [idx_v[pl.ds(i, _LANES)], zero])
            e16 = plsc.load_gather(e_v, [ii, zero])
            plsc.store_scatter(e_v, [ii, zero], e16 + r16)

        pltpu.sync_copy(e_v, out_hbm.at[pl.ds(base, chunk), :])

    return sc_kernel


def kernel(atomwise_energies, atoms, atom_refs):
    n = atomwise_energies.shape[0]
    num_types = atom_refs.shape[0]
    return _build_sc_kernel(n, num_types)(atomwise_energies, atoms, atom_refs)


# speedup vs baseline: 7.3154x; 7.3154x over previous
import functools

import jax
import jax.numpy as jnp
from jax import lax
from jax.experimental import pallas as pl
from jax.experimental.pallas import tpu as pltpu
from jax.experimental.pallas import tpu_sc as plsc

_LANES = 16


@functools.cache
def _build_sc_kernel(n: int, num_types: int):
    info = plsc.get_sparse_core_info()
    nc, ns = info.num_cores, info.num_subcores
    nw = nc * ns
    chunk = ((n + nw - 1) // nw + _LANES - 1) // _LANES * _LANES
    assert n % 8 == 0 and chunk % 8 == 0 and n >= chunk

    mesh = plsc.VectorSubcoreMesh(core_axis_name="c", subcore_axis_name="s")

    @functools.partial(
        pl.kernel,
        mesh=mesh,
        out_type=jax.ShapeDtypeStruct((n,), jnp.float32),
        compiler_params=pltpu.CompilerParams(
            needs_layout_passes=False,
            disable_bounds_checks=True,
            disable_semaphore_checks=True,
            skip_device_barrier=True,
        ),
        scratch_types=[
            pltpu.VMEM((chunk,), jnp.int32),
            pltpu.VMEM((chunk,), jnp.float32),
            pltpu.VMEM((num_types,), jnp.float32),
            pltpu.SemaphoreType.DMA,
            pltpu.SemaphoreType.DMA,
            pltpu.SemaphoreType.DMA,
        ],
    )
    def sc_kernel(e_hbm, atoms_hbm, refs_hbm, out_hbm, idx_v, e_v, tbl_v,
                  sem_i, sem_e, sem_t):
        wid = lax.axis_index("s") * nc + lax.axis_index("c")
        base = jnp.minimum(wid * chunk, n - chunk)
        base = pl.multiple_of(base, 8)
        cp_t = pltpu.async_copy(refs_hbm, tbl_v, sem_t)
        cp_i = pltpu.async_copy(atoms_hbm.at[pl.ds(base, chunk)], idx_v, sem_i)
        cp_e = pltpu.async_copy(e_hbm.at[pl.ds(base, chunk)], e_v, sem_e)
        cp_t.wait()
        cp_i.wait()
        cp_e.wait()

        @plsc.parallel_loop(0, chunk, _LANES, unroll=8)
        def _(i):
            sl = pl.ds(i, _LANES)
            e_v[sl] = e_v[sl] + plsc.load_gather(tbl_v, [idx_v[sl]])

        pltpu.sync_copy(e_v, out_hbm.at[pl.ds(base, chunk)])

    return sc_kernel


def kernel(atomwise_energies, atoms, atom_refs):
    n, t = atomwise_energies.shape
    num_types = atom_refs.shape[0]
    e_flat = atomwise_energies.reshape(n)
    refs_flat = atom_refs.reshape(num_types)
    out = _build_sc_kernel(n, num_types)(e_flat, atoms, refs_flat)
    return out.reshape(n, t)


# speedup vs baseline: 7.3241x; 1.0012x over previous
import functools

import jax
import jax.numpy as jnp
from jax import lax
from jax.experimental import pallas as pl
from jax.experimental.pallas import tpu as pltpu
from jax.experimental.pallas import tpu_sc as plsc

_LANES = 16


@functools.cache
def _build_sc_kernel(n: int, num_types: int):
    info = plsc.get_sparse_core_info()
    nc, ns = info.num_cores, info.num_subcores
    nw = nc * ns
    chunk = ((n + nw - 1) // nw + _LANES - 1) // _LANES * _LANES
    assert n % 8 == 0 and chunk % 8 == 0 and n >= chunk

    mesh = plsc.VectorSubcoreMesh(core_axis_name="c", subcore_axis_name="s")

    @functools.partial(
        pl.kernel,
        mesh=mesh,
        out_type=jax.ShapeDtypeStruct((n,), jnp.float32),
        compiler_params=pltpu.CompilerParams(
            needs_layout_passes=False,
            disable_bounds_checks=True,
            disable_semaphore_checks=True,
        ),
        scratch_types=[
            pltpu.VMEM((chunk,), jnp.int32),
            pltpu.VMEM((chunk,), jnp.float32),
            pltpu.VMEM((num_types,), jnp.float32),
            pltpu.SemaphoreType.DMA,
            pltpu.SemaphoreType.DMA,
            pltpu.SemaphoreType.DMA,
        ],
    )
    def sc_kernel(e_hbm, atoms_hbm, refs_hbm, out_hbm, idx_v, e_v, tbl_v,
                  sem_i, sem_e, sem_t):
        wid = lax.axis_index("s") * nc + lax.axis_index("c")
        base = jnp.minimum(wid * chunk, n - chunk)
        base = pl.multiple_of(base, 8)
        cp_t = pltpu.async_copy(refs_hbm, tbl_v, sem_t)
        cp_i = pltpu.async_copy(atoms_hbm.at[pl.ds(base, chunk)], idx_v, sem_i)
        cp_e = pltpu.async_copy(e_hbm.at[pl.ds(base, chunk)], e_v, sem_e)
        cp_t.wait()
        cp_i.wait()
        cp_e.wait()

        @plsc.parallel_loop(0, chunk, _LANES, unroll=8)
        def _(i):
            sl = pl.ds(i, _LANES)
            e_v[sl] = e_v[sl] + plsc.load_gather(tbl_v, [idx_v[sl]])

        pltpu.sync_copy(e_v, out_hbm.at[pl.ds(base, chunk)])

    return sc_kernel


def kernel(atomwise_energies, atoms, atom_refs):
    n, t = atomwise_energies.shape
    num_types = atom_refs.shape[0]
    e_flat = atomwise_energies.reshape(n)
    refs_flat = atom_refs.reshape(num_types)
    out = _build_sc_kernel(n, num_types)(e_flat, atoms, refs_flat)
    return out.reshape(n, t)


# speedup vs baseline: 8.0580x; 1.1002x over previous
import functools

import jax
import jax.numpy as jnp
from jax import lax
from jax.experimental import pallas as pl
from jax.experimental.pallas import tpu as pltpu
from jax.experimental.pallas import tpu_sc as plsc

_LANES = 16


@functools.cache
def _build_sc_kernel(n: int, num_types: int):
    info = plsc.get_sparse_core_info()
    nc, ns = info.num_cores, info.num_subcores
    nw = nc * ns
    chunk = ((n + nw - 1) // nw + _LANES - 1) // _LANES * _LANES
    assert n % 8 == 0 and chunk % 8 == 0 and n >= chunk

    mesh = plsc.VectorSubcoreMesh(
        core_axis_name="c", subcore_axis_name="s", num_cores=1)

    @functools.partial(
        pl.kernel,
        mesh=mesh,
        out_type=jax.ShapeDtypeStruct((n,), jnp.float32),
        compiler_params=pltpu.CompilerParams(
            needs_layout_passes=False,
            disable_bounds_checks=True,
            disable_semaphore_checks=True,
        ),
        scratch_types=[
            pltpu.VMEM((chunk,), jnp.int32),
            pltpu.VMEM((chunk,), jnp.float32),
            pltpu.VMEM((num_types,), jnp.float32),
            pltpu.SemaphoreType.DMA,
            pltpu.SemaphoreType.DMA,
            pltpu.SemaphoreType.DMA,
        ],
    )
    def sc_kernel(e_hbm, atoms_hbm, refs_hbm, out_hbm, idx_v, e_v, tbl_v,
                  sem_i, sem_e, sem_t):
        wid = lax.axis_index("s") * nc + lax.axis_index("c")
        base = jnp.minimum(wid * chunk, n - chunk)
        base = pl.multiple_of(base, 8)
        cp_t = pltpu.async_copy(refs_hbm, tbl_v, sem_t)
        cp_i = pltpu.async_copy(atoms_hbm.at[pl.ds(base, chunk)], idx_v, sem_i)
        cp_e = pltpu.async_copy(e_hbm.at[pl.ds(base, chunk)], e_v, sem_e)
        cp_t.wait()
        cp_i.wait()
        cp_e.wait()

        @plsc.parallel_loop(0, chunk, _LANES, unroll=8)
        def _(i):
            sl = pl.ds(i, _LANES)
            e_v[sl] = e_v[sl] + plsc.load_gather(tbl_v, [idx_v[sl]])

        pltpu.sync_copy(e_v, out_hbm.at[pl.ds(base, chunk)])

    return sc_kernel


def kernel(atomwise_energies, atoms, atom_refs):
    n, t = atomwise_energies.shape
    num_types = atom_refs.shape[0]
    e_flat = atomwise_energies.reshape(n)
    refs_flat = atom_refs.reshape(num_types)
    out = _build_sc_kernel(n, num_types)(e_flat, atoms, refs_flat)
    return out.reshape(n, t)
